# baseline (speedup 1.0000x reference)
import functools

import jax
import jax.numpy as jnp
from jax import lax
from jax.experimental import pallas as pl
from jax.experimental.pallas import tpu as pltpu
from jax.experimental.pallas import tpu_sc as plsc

N = 10000
NPAD = 10240
E = 320000
D = 128
ALPHA = 0.2
NS = 16
SCHUNK = 128
ER = E // SCHUNK
ERP = 2560
RF = ERP // NS
CN = NPAD // NS

_f32 = jnp.float32
_sds = jax.ShapeDtypeStruct
_sc_params = pltpu.CompilerParams(needs_layout_passes=False)
_mesh1 = plsc.VectorSubcoreMesh(
    core_axis_name="c", subcore_axis_name="s", num_cores=1
)


def _fill(ref, n_vecs, value):
    v = jnp.full((16,), value, _f32)

    @plsc.parallel_loop(0, n_vecs, unroll=4)
    def _(i):
        ref[pl.ds(i * 16, 16)] = v


def _stage(pairs, sem):
    descs = [pltpu.async_copy(s, d, sem) for s, d in pairs]
    for dsc in descs:
        dsc.wait()


def _rsqrt16(x):
    i = lax.bitcast_convert_type(x, jnp.int32)
    i = jnp.int32(0x5F3759DF) - (i >> 1)
    y = lax.bitcast_convert_type(i, _f32)
    y = y * (1.5 - 0.5 * x * y * y)
    y = y * (1.5 - 0.5 * x * y * y)
    y = y * (1.5 - 0.5 * x * y * y)
    return y


def _h0_body(x_ref, w_ref, b_ref, o_ref):
    o_ref[...] = (
        jnp.dot(x_ref[...], w_ref[...], preferred_element_type=_f32)[:, 0]
        + b_ref[0]
    )


def _h0_tc(x, W0, b0):
    blk = NPAD // 10
    return pl.pallas_call(
        _h0_body,
        grid=(10,),
        in_specs=[
            pl.BlockSpec((blk, D), lambda i: (i, 0)),
            pl.BlockSpec((D, 1), lambda i: (0, 0)),
            pl.BlockSpec(memory_space=pltpu.SMEM),
        ],
        out_specs=pl.BlockSpec((blk,), lambda i: (i,)),
        out_shape=_sds((NPAD,), _f32),
    )(x, W0, b0)


@functools.partial(
    pl.kernel,
    out_type=(_sds((N,), _f32), _sds((NPAD,), _f32)),
    mesh=_mesh1,
    compiler_params=_sc_params,
    scratch_types=[
        pltpu.VMEM((RF, SCHUNK), jnp.int32),
        pltpu.VMEM((RF, SCHUNK), jnp.int32),
        pltpu.VMEM((RF, SCHUNK), _f32),
        pltpu.VMEM((NPAD,), _f32),
        pltpu.VMEM((SCHUNK,), _f32),
        pltpu.VMEM((CN,), _f32),
        pltpu.VMEM((CN,), _f32),
        pltpu.VMEM((CN,), _f32),
        pltpu.VMEM((CN,), _f32),
        pltpu.VMEM((64,), _f32),
        pltpu.VMEM_SHARED((NPAD,), _f32),
        pltpu.SemaphoreType.DMA,
    ],
)
def _mega_kernel(ei3, h0p, wb, y, u_hbm,
                 src2_v, dst2_v, msg2_v, u_v, ones_v, d_c, h_c, s_c,
                 zero_v, wb_v, agg_sp, sem):
    sid = lax.axis_index("s")
    chunk = pl.ds(sid * CN, CN)
    rowbase = sid * RF

    _stage(
        [
            (ei3.at[0].at[pl.ds(rowbase, RF)], src2_v),
            (ei3.at[1].at[pl.ds(rowbase, RF)], dst2_v),
            (h0p.at[chunk], h_c),
            (wb, wb_v),
        ],
        sem,
    )
    _fill(ones_v, SCHUNK // 16, 1.0)
    _fill(zero_v, CN // 16, 0.0)
    pltpu.sync_copy(zero_v, agg_sp.at[chunk])
    plsc.subcore_barrier()

    def degf(g, _):
        base = g * 20
        for j in range(20):
            pltpu.async_copy(
                ones_v, agg_sp.at[dst2_v.at[base + j]], sem, add=True
            )
        return 0

    lax.fori_loop(0, RF // 20, degf, 0)

    def degd(g, _):
        base = g * 20
        for j in range(20):
            pltpu.make_async_copy(
                ones_v, agg_sp.at[dst2_v.at[base + j]], sem
            ).wait()
        return 0

    lax.fori_loop(0, RF // 20, degd, 0)
    plsc.subcore_barrier()
    pltpu.sync_copy(agg_sp.at[chunk], s_c)

    @plsc.parallel_loop(0, CN // 16, unroll=4)
    def _(i):
        o = pl.ds(i * 16, 16)
        dd = _rsqrt16(s_c[o] + 1.0)
        d_c[o] = dd
        s_c[o] = dd * h_c[o]

    pltpu.sync_copy(s_c, u_hbm.at[chunk])
    pltpu.sync_copy(zero_v, agg_sp.at[chunk])
    plsc.subcore_barrier()

    for r in (1, 2, 3):
        pltpu.sync_copy(u_hbm, u_v)
        G2 = 16

        def gs(g, _):
            base = g * G2

            @plsc.parallel_loop(base, base + G2, unroll=2)
            def _(rr):
                for c in range(8):
                    o = pl.ds(c * 16, 16)
                    msg2_v[rr, o] = plsc.load_gather(u_v, [src2_v[rr, o]])

            for j in range(G2):
                pltpu.async_copy(
                    msg2_v.at[base + j],
                    agg_sp.at[dst2_v.at[base + j]],
                    sem,
                    add=True,
                )
            return 0

        lax.fori_loop(0, RF // G2, gs, 0)

        def drain(g, _):
            base = g * G2
            for j in range(G2):
                pltpu.make_async_copy(
                    msg2_v.at[base + j],
                    agg_sp.at[dst2_v.at[base + j]],
                    sem,
                ).wait()
            return 0

        lax.fori_loop(0, RF // G2, drain, 0)
        plsc.subcore_barrier()
        pltpu.sync_copy(agg_sp.at[chunk], s_c)

        if r < 3:
            wv = wb_v[pl.ds((r - 1) * 32, 16)]
            bv = wb_v[pl.ds((r - 1) * 32 + 16, 16)]

            @plsc.parallel_loop(0, CN // 16, unroll=4)
            def _(i):
                o = pl.ds(i * 16, 16)
                dd = d_c[o]
                hp = h_c[o]
                z = (1.0 - ALPHA) * (dd * s_c[o] + dd * dd * hp) + ALPHA * hp
                hr = z * wv + bv
                h_c[o] = hr
                s_c[o] = dd * hr

            pltpu.sync_copy(s_c, u_hbm.at[chunk])
            pltpu.sync_copy(zero_v, agg_sp.at[chunk])
            plsc.subcore_barrier()
        else:

            @plsc.parallel_loop(0, CN // 16, unroll=4)
            def _(i):
                o = pl.ds(i * 16, 16)
                dd = d_c[o]
                hp = h_c[o]
                z = (1.0 - ALPHA) * (dd * s_c[o] + dd * dd * hp) + ALPHA * hp
                s_c[o] = jnp.maximum(z, 0.0) + 0.001

            @pl.when(sid < 15)
            def _():
                pltpu.sync_copy(s_c, y.at[pl.ds(sid * CN, CN)])

            @pl.when(sid == 15)
            def _():
                pltpu.sync_copy(
                    s_c.at[pl.ds(0, N - 15 * CN)],
                    y.at[pl.ds(15 * CN, N - 15 * CN)],
                )


def _wb_vec(W, b):
    return jnp.concatenate(
        [
            jnp.broadcast_to(W.reshape(-1)[:1], (16,)),
            jnp.broadcast_to(b.reshape(-1)[:1], (16,)),
        ]
    ).astype(_f32)


def kernel(x, edge_index, W0, b0, W1, b1, W2, b2):
    padc = (jnp.arange((ERP - ER) * SCHUNK, dtype=jnp.int32) % (NPAD - N) + N
            ).reshape(1, ERP - ER, SCHUNK)
    ei3 = jnp.concatenate(
        [
            edge_index.astype(jnp.int32).reshape(2, ER, SCHUNK),
            jnp.broadcast_to(padc, (2, ERP - ER, SCHUNK)),
        ],
        axis=1,
    )
    h0p = _h0_tc(x, W0, b0)
    wb = jnp.concatenate([_wb_vec(W1, b1), _wb_vec(W2, b2)])
    y, _unused = _mega_kernel(ei3, h0p, wb)
    return y.reshape(N, 1)
